# baseline (speedup 1.0000x reference)
import functools

import jax
import jax.numpy as jnp
from jax import lax
from jax.experimental import pallas as pl
from jax.experimental.pallas import tpu as pltpu
from jax.experimental.pallas import tpu_sc as plsc

T = 16384
H = 4096
E = 64
K = 8
EPS = 1e-05

BT = 1024
SIZES = (12288, 4096)
STARTS = (0, 12288)
NW = 32


def _dense_compute(x_ref, scale_ref, w_ref, b_ref, t_ref, g_ref):
    x32 = x_ref[...].astype(jnp.float32)
    ssq = jnp.sum(x32 * x32, axis=1, keepdims=True)
    rstd = lax.rsqrt(ssq * (1.0 / H) + EPS)
    t = ((x32 * rstd) * scale_ref[...]).astype(jnp.bfloat16)
    t_ref[...] = t
    g32 = lax.dot_general(t, w_ref[...], (((1,), (1,)), ((), ())),
                          preferred_element_type=jnp.float32)
    g_ref[...] = g32 + b_ref[...].astype(jnp.float32)


def _dense_body0(x_ref, scale_ref, w_ref, b_ref, t_ref, g_ref):
    _dense_compute(x_ref, scale_ref, w_ref, b_ref, t_ref, g_ref)


def _dense_bodyn(x_ref, scale_ref, w_ref, b_ref, ti_ref, t_ref, g_ref):
    del ti_ref
    _dense_compute(x_ref, scale_ref, w_ref, b_ref, t_ref, g_ref)


def _dense_chunk(start, ct, x, scale2d, gate_w, gate_b2d, t_prev):
    b0 = start // BT
    in_specs = [
        pl.BlockSpec((BT, H), lambda i, b0=b0: (b0 + i, 0)),
        pl.BlockSpec((1, H), lambda i: (0, 0)),
        pl.BlockSpec((E, H), lambda i: (0, 0)),
        pl.BlockSpec((1, E), lambda i: (0, 0)),
    ]
    args = [x, scale2d, gate_w, gate_b2d]
    if t_prev is None:
        body = _dense_body0
        io_alias = {}
    else:
        body = _dense_bodyn
        in_specs.append(pl.BlockSpec((8, 128), lambda i: (0, 0)))
        args.append(t_prev)
        io_alias = {4: 0}
    return pl.pallas_call(
        body,
        grid=(ct // BT,),
        in_specs=in_specs,
        out_specs=[
            pl.BlockSpec((BT, H), lambda i, b0=b0: (b0 + i, 0)),
            pl.BlockSpec((BT, E), lambda i: (i, 0)),
        ],
        out_shape=[
            jax.ShapeDtypeStruct((T, H), jnp.bfloat16),
            jax.ShapeDtypeStruct((ct, E), jnp.float32),
        ],
        input_output_aliases=io_alias,
    )(*args)


def _round_f32_to_bf16(v):
    b = plsc.bitcast(v, jnp.uint32)
    r = (b + jnp.uint32(0x7FFF) + ((b >> jnp.uint32(16)) & jnp.uint32(1)))
    return plsc.bitcast(r & jnp.uint32(0xFFFF0000), jnp.float32)


def _sc_topk(g_flat, ct):
    mesh = plsc.VectorSubcoreMesh(core_axis_name="c", subcore_axis_name="s")
    rows = ct // NW

    @functools.partial(
        pl.kernel,
        mesh=mesh,
        compiler_params=pltpu.CompilerParams(needs_layout_passes=False),
        out_type=(
            jax.ShapeDtypeStruct((ct * 16,), jnp.float32),
            jax.ShapeDtypeStruct((ct * 16,), jnp.int32),
        ),
        scratch_types=[
            pltpu.VMEM((rows * E,), jnp.float32),
            pltpu.VMEM((rows * 16,), jnp.float32),
            pltpu.VMEM((rows * 16,), jnp.int32),
        ],
    )
    def k(g_hbm, w_hbm, i_hbm, g_v, w_v, i_v):
        ROWS = rows
        wid = lax.axis_index("s") * 2 + lax.axis_index("c")
        base = wid * ROWS
        pltpu.sync_copy(g_hbm.at[pl.ds(base * E, ROWS * E)], g_v)

        lane = lax.iota(jnp.int32, 16)
        lane_u = lane.astype(jnp.uint32)

        def merge(a, b):
            rb = lax.rev(b, (0,))
            l = jnp.where(a >= rb, a, rb)
            return plsc.sort_key_val(l, l, descending=True)[0]

        @plsc.parallel_loop(0, ROWS, 1, unroll=4)
        def body(tok):
            srt = []
            for j in range(4):
                v = g_v[pl.ds(tok * E + j * 16, 16)]
                bits = plsc.bitcast(v, jnp.uint32)
                mono = jnp.where(v < 0.0, ~bits, bits ^ jnp.uint32(0x80000000))
                key = (mono & jnp.uint32(0xFFFF0000)) | (
                    jnp.uint32(63 - j * 16) - lane_u)
                srt.append(plsc.sort_key_val(key, key, descending=True)[0])
            top = merge(merge(srt[0], srt[1]), merge(srt[2], srt[3]))

            idxv = (jnp.uint32(63) - (top & jnp.uint32(0xFFFF))).astype(jnp.int32)
            vtop = top & jnp.uint32(0xFFFF0000)
            pos = top >= jnp.uint32(0x80000000)
            vbits = jnp.where(pos, vtop ^ jnp.uint32(0x80000000), ~vtop)
            vals = plsc.bitcast(vbits & jnp.uint32(0xFFFF0000), jnp.float32)

            d = _round_f32_to_bf16(vals - jnp.max(vals))
            e = _round_f32_to_bf16(jnp.exp(d))
            s = jnp.sum(jnp.where(lane < 8, e, 0.0))
            sv = _round_f32_to_bf16(jnp.broadcast_to(s, (16,)))
            w_v[pl.ds(tok * 16, 16)] = e / sv
            i_v[pl.ds(tok * 16, 16)] = idxv

        pltpu.sync_copy(w_v, w_hbm.at[pl.ds(base * 16, ROWS * 16)])
        pltpu.sync_copy(i_v, i_hbm.at[pl.ds(base * 16, ROWS * 16)])

    return k(g_flat)


def kernel(x, scale, gate_w, gate_b):
    scale2d = scale.reshape(1, H)
    gate_b2d = gate_b.reshape(1, E)
    t = None
    ws, idxs = [], []
    for start, ct in zip(STARTS, SIZES):
        t, g = _dense_chunk(start, ct, x, scale2d, gate_w, gate_b2d, t)
        w16, i16 = _sc_topk(g.reshape(ct * E), ct)
        ws.append(w16.reshape(ct, 16)[:, :K])
        idxs.append(i16.reshape(ct, 16)[:, :K])
    ew = jnp.concatenate(ws).astype(jnp.bfloat16)
    idx = jnp.concatenate(idxs)
    return (t, ew, idx)


# speedup vs baseline: 1.0680x; 1.0680x over previous
import functools

import jax
import jax.numpy as jnp
from jax import lax
from jax.experimental import pallas as pl
from jax.experimental.pallas import tpu as pltpu
from jax.experimental.pallas import tpu_sc as plsc

T = 16384
H = 4096
E = 64
K = 8
EPS = 1e-05

BT = 1024
SIZES = (8192, 8192)
STARTS = (0, 8192)
NW = 32


def _dense_compute(x_ref, scale_ref, w_ref, b_ref, t_ref, g_ref):
    x32 = x_ref[...].astype(jnp.float32)
    ssq = jnp.sum(x32 * x32, axis=1, keepdims=True)
    rstd = lax.rsqrt(ssq * (1.0 / H) + EPS)
    t = ((x32 * rstd) * scale_ref[...]).astype(jnp.bfloat16)
    t_ref[...] = t
    g32 = lax.dot_general(t, w_ref[...], (((1,), (1,)), ((), ())),
                          preferred_element_type=jnp.float32)
    g_ref[...] = g32 + b_ref[...].astype(jnp.float32)


def _dense_body0(x_ref, scale_ref, w_ref, b_ref, t_ref, g_ref):
    _dense_compute(x_ref, scale_ref, w_ref, b_ref, t_ref, g_ref)


def _dense_bodyn(x_ref, scale_ref, w_ref, b_ref, ti_ref, t_ref, g_ref):
    del ti_ref
    _dense_compute(x_ref, scale_ref, w_ref, b_ref, t_ref, g_ref)


def _dense_chunk(start, ct, x, scale2d, gate_w, gate_b2d, t_prev):
    b0 = start // BT
    in_specs = [
        pl.BlockSpec((BT, H), lambda i, b0=b0: (b0 + i, 0)),
        pl.BlockSpec((1, H), lambda i: (0, 0)),
        pl.BlockSpec((E, H), lambda i: (0, 0)),
        pl.BlockSpec((1, E), lambda i: (0, 0)),
    ]
    args = [x, scale2d, gate_w, gate_b2d]
    if t_prev is None:
        body = _dense_body0
        io_alias = {}
    else:
        body = _dense_bodyn
        in_specs.append(pl.BlockSpec((8, 128), lambda i: (0, 0)))
        args.append(t_prev)
        io_alias = {4: 0}
    return pl.pallas_call(
        body,
        grid=(ct // BT,),
        in_specs=in_specs,
        out_specs=[
            pl.BlockSpec((BT, H), lambda i, b0=b0: (b0 + i, 0)),
            pl.BlockSpec((BT, E), lambda i: (i, 0)),
        ],
        out_shape=[
            jax.ShapeDtypeStruct((T, H), jnp.bfloat16),
            jax.ShapeDtypeStruct((ct, E), jnp.float32),
        ],
        input_output_aliases=io_alias,
    )(*args)


def _round_f32_to_bf16(v):
    b = plsc.bitcast(v, jnp.uint32)
    r = (b + jnp.uint32(0x7FFF) + ((b >> jnp.uint32(16)) & jnp.uint32(1)))
    return plsc.bitcast(r & jnp.uint32(0xFFFF0000), jnp.float32)


def _sc_topk(g_flat, ct):
    mesh = plsc.VectorSubcoreMesh(core_axis_name="c", subcore_axis_name="s")
    rows = ct // NW

    @functools.partial(
        pl.kernel,
        mesh=mesh,
        compiler_params=pltpu.CompilerParams(needs_layout_passes=False),
        out_type=(
            jax.ShapeDtypeStruct((ct * 16,), jnp.float32),
            jax.ShapeDtypeStruct((ct * 16,), jnp.int32),
        ),
        scratch_types=[
            pltpu.VMEM((rows * E,), jnp.float32),
            pltpu.VMEM((rows * 16,), jnp.float32),
            pltpu.VMEM((rows * 16,), jnp.int32),
        ],
    )
    def k(g_hbm, w_hbm, i_hbm, g_v, w_v, i_v):
        ROWS = rows
        wid = lax.axis_index("s") * 2 + lax.axis_index("c")
        base = wid * ROWS
        pltpu.sync_copy(g_hbm.at[pl.ds(base * E, ROWS * E)], g_v)

        lane = lax.iota(jnp.int32, 16)
        lane_u = lane.astype(jnp.uint32)

        def merge(a, b):
            rb = lax.rev(b, (0,))
            l = jnp.where(a >= rb, a, rb)
            return plsc.sort_key_val(l, l, descending=True)[0]

        @plsc.parallel_loop(0, ROWS, 1, unroll=4)
        def body(tok):
            srt = []
            for j in range(4):
                v = g_v[pl.ds(tok * E + j * 16, 16)]
                bits = plsc.bitcast(v, jnp.uint32)
                mono = jnp.where(v < 0.0, ~bits, bits ^ jnp.uint32(0x80000000))
                key = (mono & jnp.uint32(0xFFFF0000)) | (
                    jnp.uint32(63 - j * 16) - lane_u)
                srt.append(plsc.sort_key_val(key, key, descending=True)[0])
            top = merge(merge(srt[0], srt[1]), merge(srt[2], srt[3]))

            idxv = (jnp.uint32(63) - (top & jnp.uint32(0xFFFF))).astype(jnp.int32)
            vtop = top & jnp.uint32(0xFFFF0000)
            pos = top >= jnp.uint32(0x80000000)
            vbits = jnp.where(pos, vtop ^ jnp.uint32(0x80000000), ~vtop)
            vals = plsc.bitcast(vbits & jnp.uint32(0xFFFF0000), jnp.float32)

            d = _round_f32_to_bf16(vals - jnp.max(vals))
            e = _round_f32_to_bf16(jnp.exp(d))
            s = jnp.sum(jnp.where(lane < 8, e, 0.0))
            sv = _round_f32_to_bf16(jnp.broadcast_to(s, (16,)))
            w_v[pl.ds(tok * 16, 16)] = e / sv
            i_v[pl.ds(tok * 16, 16)] = idxv

        pltpu.sync_copy(w_v, w_hbm.at[pl.ds(base * 16, ROWS * 16)])
        pltpu.sync_copy(i_v, i_hbm.at[pl.ds(base * 16, ROWS * 16)])

    return k(g_flat)


def kernel(x, scale, gate_w, gate_b):
    scale2d = scale.reshape(1, H)
    gate_b2d = gate_b.reshape(1, E)
    t = None
    ws, idxs = [], []
    for start, ct in zip(STARTS, SIZES):
        t, g = _dense_chunk(start, ct, x, scale2d, gate_w, gate_b2d, t)
        w16, i16 = _sc_topk(g.reshape(ct * E), ct)
        ws.append(w16.reshape(ct, 16)[:, :K])
        idxs.append(i16.reshape(ct, 16)[:, :K])
    ew = jnp.concatenate(ws).astype(jnp.bfloat16)
    idx = jnp.concatenate(idxs)
    return (t, ew, idx)
